# baseline (speedup 1.0000x reference)
import jax
import jax.numpy as jnp
from jax.experimental import pallas as pl
from jax.experimental.pallas import tpu as pltpu
from jax.experimental.pallas import tpu_sc as plsc

NUM_CODES = 8192
DIM = 32
ROWS_PER_BLOCK = 128
GATHER_WINDOW = 128


CHUNK = 128


def _vq_block_kernel(x_ref, xm2_ref, w_ref, wt_ref, ids_ref, loss_ref, w2_ref):
    @pl.when(pl.program_id(0) == 0)
    def _():
        w = w_ref[...]
        w2_ref[0, :] = jnp.sum(w * w, axis=-1)

    R = x_ref.shape[0]
    x = x_ref[...]
    xm2 = xm2_ref[...]
    x2b = jnp.sum(x * x, axis=-1, keepdims=True)

    best = jnp.full((R, CHUNK), jnp.inf, jnp.float32)
    bestchunk = jnp.zeros((R, CHUNK), jnp.int32)
    for c in range(NUM_CODES // CHUNK):
        wt_c = wt_ref[:, pl.ds(c * CHUNK, CHUNK)]
        mm_c = jax.lax.dot_general(xm2, wt_c, (((1,), (0,)), ((), ())),
                                   preferred_element_type=jnp.float32)
        d_c = (x2b + mm_c) + w2_ref[:, pl.ds(c * CHUNK, CHUNK)]
        upd = d_c < best
        best = jnp.where(upd, d_c, best)
        bestchunk = jnp.where(upd, c, bestchunk)

    lane = jax.lax.broadcasted_iota(jnp.int32, (R, CHUNK), 1)
    bid = bestchunk * CHUNK + lane
    m = jnp.min(best, axis=-1, keepdims=True)
    ids = jnp.min(jnp.where(best == m, bid, NUM_CODES), axis=-1,
                  keepdims=True)
    ids_ref[...] = ids
    loss_ref[...] = m * (1.0 / DIM)


def _distance_argmin(x, W):
    n = x.shape[0]
    R = ROWS_PER_BLOCK
    G = n // R
    ids3, loss3 = pl.pallas_call(
        _vq_block_kernel,
        grid=(G,),
        in_specs=[
            pl.BlockSpec((R, DIM), lambda i: (i, 0)),
            pl.BlockSpec((R, DIM), lambda i: (i, 0)),
            pl.BlockSpec((NUM_CODES, DIM), lambda i: (0, 0)),
            pl.BlockSpec((DIM, NUM_CODES), lambda i: (0, 0)),
        ],
        out_specs=[
            pl.BlockSpec((R, 1), lambda i: (i, 0)),
            pl.BlockSpec((R, 1), lambda i: (i, 0)),
        ],
        out_shape=[
            jax.ShapeDtypeStruct((n, 1), jnp.int32),
            jax.ShapeDtypeStruct((n, 1), jnp.float32),
        ],
        scratch_shapes=[pltpu.VMEM((1, NUM_CODES), jnp.float32)],
    )(x, x * -2.0, W, W.T)
    return ids3.reshape(n), loss3.reshape(n)


GATHER_LANES = 128


def _sc_gather(Wp, ids_flat):
    n = ids_flat.shape[0]
    idx2 = ids_flat.reshape(1, n)
    mesh = plsc.VectorSubcoreMesh(core_axis_name="core",
                                  subcore_axis_name="subcore")

    @pl.kernel(out_type=jax.ShapeDtypeStruct((n, GATHER_LANES), Wp.dtype),
               mesh=mesh)
    def kern(w_hbm, i_hbm, o_hbm):
        def body(i_vmem, o_vmem):
            pltpu.sync_copy(w_hbm.at[i_vmem.at[0]], o_vmem)

        pltpu.emit_pipeline(
            body,
            grid=(n // GATHER_WINDOW,),
            in_specs=[pl.BlockSpec((1, GATHER_WINDOW),
                                   index_map=lambda i: (0, i))],
            out_specs=[pl.BlockSpec((GATHER_WINDOW, GATHER_LANES),
                                    index_map=lambda i: (i, 0))],
            core_axis_name=("core", "subcore"),
            dimension_semantics=(pltpu.PARALLEL,),
        )(i_hbm, o_hbm)

    return kern(Wp, idx2)


def kernel(inp, W):
    B, S, D = inp.shape
    n = B * S
    x = inp.reshape(n, D)
    ids_flat, loss_flat = _distance_argmin(x, W)
    Wp = jnp.pad(W, ((0, 0), (0, GATHER_LANES - D)))
    q = _sc_gather(Wp, ids_flat)[:, :DIM]
    ids = ids_flat.reshape(B, S)
    loss = loss_flat.reshape(B, S)
    quantized = q.reshape(B, S, D)
    losses = {'commitment': loss, 'codebook': loss}
    return (quantized, ids, losses)


# speedup vs baseline: 1.1293x; 1.1293x over previous
import jax
import jax.numpy as jnp
from jax.experimental import pallas as pl
from jax.experimental.pallas import tpu as pltpu
from jax.experimental.pallas import tpu_sc as plsc

NUM_CODES = 8192
DIM = 32
ROWS_PER_BLOCK = 256
GATHER_WINDOW = 128


CHUNK = 128


def _vq_block_kernel(x_ref, w_ref, wt_ref, ids_ref, loss_ref, w2_ref):
    @pl.when(pl.program_id(0) == 0)
    def _():
        w = w_ref[...]
        w2_ref[0, :] = jnp.sum(w * w, axis=-1)

    R = x_ref.shape[0]
    x = x_ref[...]
    xm2 = x * -2.0
    x2b = jnp.sum(x * x, axis=-1, keepdims=True)

    best = jnp.full((R, CHUNK), jnp.inf, jnp.float32)
    bestchunk = jnp.zeros((R, CHUNK), jnp.int32)
    for c in range(NUM_CODES // CHUNK):
        wt_c = wt_ref[:, pl.ds(c * CHUNK, CHUNK)]
        mm_c = jax.lax.dot_general(xm2, wt_c, (((1,), (0,)), ((), ())),
                                   preferred_element_type=jnp.float32)
        d_c = (x2b + mm_c) + w2_ref[:, pl.ds(c * CHUNK, CHUNK)]
        upd = d_c < best
        best = jnp.where(upd, d_c, best)
        bestchunk = jnp.where(upd, c, bestchunk)

    lane = jax.lax.broadcasted_iota(jnp.int32, (R, CHUNK), 1)
    bid = bestchunk * CHUNK + lane
    m = jnp.min(best, axis=-1, keepdims=True)
    ids = jnp.min(jnp.where(best == m, bid, NUM_CODES), axis=-1,
                  keepdims=True)
    ids_ref[...] = ids
    loss_ref[...] = m * (1.0 / DIM)


def _distance_argmin(x, W):
    n = x.shape[0]
    R = ROWS_PER_BLOCK
    G = n // R
    ids3, loss3 = pl.pallas_call(
        _vq_block_kernel,
        grid=(G,),
        in_specs=[
            pl.BlockSpec((R, DIM), lambda i: (i, 0)),
            pl.BlockSpec((NUM_CODES, DIM), lambda i: (0, 0)),
            pl.BlockSpec((DIM, NUM_CODES), lambda i: (0, 0)),
        ],
        out_specs=[
            pl.BlockSpec((R, 1), lambda i: (i, 0)),
            pl.BlockSpec((R, 1), lambda i: (i, 0)),
        ],
        out_shape=[
            jax.ShapeDtypeStruct((n, 1), jnp.int32),
            jax.ShapeDtypeStruct((n, 1), jnp.float32),
        ],
        scratch_shapes=[pltpu.VMEM((1, NUM_CODES), jnp.float32)],
    )(x, W, W.T)
    return ids3.reshape(n), loss3.reshape(n)


GATHER_LANES = 128


def _sc_gather(Wp, ids_flat):
    n = ids_flat.shape[0]
    idx2 = ids_flat.reshape(1, n)
    mesh = plsc.VectorSubcoreMesh(core_axis_name="core",
                                  subcore_axis_name="subcore")

    @pl.kernel(out_type=jax.ShapeDtypeStruct((n, GATHER_LANES), Wp.dtype),
               mesh=mesh)
    def kern(w_hbm, i_hbm, o_hbm):
        def body(i_vmem, o_vmem):
            pltpu.sync_copy(w_hbm.at[i_vmem.at[0]], o_vmem)

        pltpu.emit_pipeline(
            body,
            grid=(n // GATHER_WINDOW,),
            in_specs=[pl.BlockSpec((1, GATHER_WINDOW),
                                   index_map=lambda i: (0, i))],
            out_specs=[pl.BlockSpec((GATHER_WINDOW, GATHER_LANES),
                                    index_map=lambda i: (i, 0))],
            core_axis_name=("core", "subcore"),
            dimension_semantics=(pltpu.PARALLEL,),
        )(i_hbm, o_hbm)

    return kern(Wp, idx2)


def kernel(inp, W):
    B, S, D = inp.shape
    n = B * S
    x = inp.reshape(n, D)
    ids_flat, loss_flat = _distance_argmin(x, W)
    Wp = jnp.pad(W, ((0, 0), (0, GATHER_LANES - D)))
    q = _sc_gather(Wp, ids_flat)[:, :DIM]
    ids = ids_flat.reshape(B, S)
    loss = loss_flat.reshape(B, S)
    quantized = q.reshape(B, S, D)
    losses = {'commitment': loss, 'codebook': loss}
    return (quantized, ids, losses)
